# baseline (speedup 1.0000x reference)
import functools

import jax
import jax.numpy as jnp
from jax import lax
from jax.experimental import pallas as pl
from jax.experimental.pallas import tpu as pltpu
from jax.experimental.pallas import tpu_sc as plsc

N = 10000
NEXT = 10240
NDUMP = NEXT - N
D = 128
E = 320000
NC, NS = 2, 16
NW = NC * NS
WIN = 64
NWINC = 176
EPW = E // NW
CAP = NWINC * WIN
STR1 = NEXT // NS
F32 = jnp.float32
I32 = jnp.int32


def _mesh():
    return plsc.VectorSubcoreMesh(
        core_axis_name="c", subcore_axis_name="s", num_cores=NC, num_subcores=NS)


def _scd_body(r_hbm, c_hbm, cnt_hbm, a_hbm, z1_hbm,
              ro_hbm, co_hbm, cnto_hbm, d0_hbm, d1_hbm,
              a_v, ridx, cidx, rout, cout, actv, cntv, cnto_v, iota_v,
              deg_sp, cnt_sp):
    cid = lax.axis_index("c")
    sid = lax.axis_index("s")
    wid = sid * NC + cid
    lane = lax.iota(I32, 16)

    @pl.when(sid == 0)
    def _():
        pltpu.sync_copy(z1_hbm.at[pl.ds(0, 128)], cnt_sp)

    pltpu.sync_copy(z1_hbm.at[pl.ds(sid * STR1, STR1)],
                    deg_sp.at[pl.ds(sid * STR1, STR1)])
    pltpu.sync_copy(a_hbm, a_v)
    pltpu.sync_copy(cnt_hbm.at[cid], cntv)
    pltpu.sync_copy(r_hbm.at[wid], ridx)
    pltpu.sync_copy(c_hbm.at[wid], cidx)
    iota_v[...] = lane
    plsc.subcore_barrier()

    mycnt = jnp.sum(jnp.where(lane == sid, cntv[pl.ds(0, 16)], 0.0)).astype(I32)
    nwin = (mycnt + (WIN - 1)) >> 6

    def win(w, off):
        for j in range(WIN // 16):
            rv = ridx[w, pl.ds(j * 16, 16)]
            cv = cidx[w, pl.ds(j * 16, 16)]
            ar = plsc.load_gather(a_v, [rv])
            ac = plsc.load_gather(a_v, [cv])
            act = (ar * ac) > 0.5
            acti = act.astype(I32)
            pos = off + plsc.cumsum(acti) - 1
            plsc.store_scatter(rout, [pos >> 6, pos & (WIN - 1)], rv, mask=act)
            plsc.store_scatter(cout, [pos >> 6, pos & (WIN - 1)], cv, mask=act)
            off = off + jnp.sum(acti)
            actv[pl.ds(j * 16, 16)] = jnp.where(act, 1.0, 0.0).astype(F32)
        pltpu.sync_copy(actv, deg_sp.at[cidx.at[w]], add=True)
        return off

    off = lax.fori_loop(0, nwin, win, jnp.int32(0))

    for j in range(WIN // 16):
        pos = off + j * 16 + lane
        dmp = N + lax.rem((wid * 61 + j) * 16 + lane, jnp.full((16,), NDUMP, I32))
        plsc.store_scatter(rout, [pos >> 6, pos & (WIN - 1)], dmp)
        plsc.store_scatter(cout, [pos >> 6, pos & (WIN - 1)], dmp)

    pltpu.sync_copy(rout, ro_hbm.at[wid])
    pltpu.sync_copy(cout, co_hbm.at[wid])
    cnto_v[...] = jnp.where(lane == sid, off.astype(F32), 0.0)
    pltpu.sync_copy(cnto_v, cnt_sp.at[iota_v], add=True)
    plsc.subcore_barrier()

    @pl.when(cid == 0)
    def _():
        pltpu.sync_copy(deg_sp.at[pl.ds(sid * STR1, STR1)],
                        d0_hbm.at[pl.ds(sid * STR1, STR1)])

    @pl.when(cid == 1)
    def _():
        pltpu.sync_copy(deg_sp.at[pl.ds(sid * STR1, STR1)],
                        d1_hbm.at[pl.ds(sid * STR1, STR1)])

    @pl.when(sid == 0)
    def _():
        pltpu.sync_copy(cnt_sp, cnto_hbm.at[cid])


def _scd(r, c, cnt, a, z1):
    k = functools.partial(
        pl.kernel, _scd_body,
        out_type=(jax.ShapeDtypeStruct((NW, NWINC, WIN), I32),
                  jax.ShapeDtypeStruct((NW, NWINC, WIN), I32),
                  jax.ShapeDtypeStruct((NC, 128), F32),
                  jax.ShapeDtypeStruct((NEXT,), F32),
                  jax.ShapeDtypeStruct((NEXT,), F32)),
        mesh=_mesh(),
        compiler_params=pltpu.CompilerParams(needs_layout_passes=False),
        scratch_types=[
            pltpu.VMEM((NEXT,), F32),
            pltpu.VMEM((NWINC, WIN), I32),
            pltpu.VMEM((NWINC, WIN), I32),
            pltpu.VMEM((NWINC, WIN), I32),
            pltpu.VMEM((NWINC, WIN), I32),
            pltpu.VMEM((WIN,), F32),
            pltpu.VMEM((128,), F32),
            pltpu.VMEM((NS,), F32),
            pltpu.VMEM((16,), I32),
            pltpu.VMEM_SHARED((NEXT,), F32),
            pltpu.VMEM_SHARED((128,), F32),
        ])()
    return k(r, c, cnt, a, z1)


def _scb_body(hs_hbm, r_hbm, c_hbm, cnt_hbm, z2_hbm, acc_hbm,
              ridx, cidx, cntv, rows_v, acc_sp, gsem, ssem):
    cid = lax.axis_index("c")
    sid = lax.axis_index("s")
    wid = sid * NC + cid
    lane = lax.iota(I32, 16)
    pltpu.sync_copy(z2_hbm.at[pl.ds(sid * STR1, STR1)],
                    acc_sp.at[pl.ds(sid * STR1, STR1)])
    pltpu.sync_copy(cnt_hbm.at[cid], cntv)
    plsc.subcore_barrier()

    mycnt = jnp.sum(jnp.where(lane == sid, cntv[pl.ds(0, 16)], 0.0)).astype(I32)
    nwin = (mycnt + (WIN - 1)) >> 6

    CW = NWINC // 2
    for ch in range(2):
        nw = jnp.clip(nwin - ch * CW, 0, CW)

        @pl.when(nw > 0)
        def _():
            pltpu.sync_copy(r_hbm.at[wid, pl.ds(ch * CW, CW)], ridx)
            pltpu.sync_copy(c_hbm.at[wid, pl.ds(ch * CW, CW)], cidx)
            pltpu.async_copy(hs_hbm.at[ridx.at[0]], rows_v.at[0], gsem.at[0])

            def win(w, carry):
                b = lax.rem(w, 2)
                pltpu.make_async_copy(hs_hbm.at[ridx.at[w]], rows_v.at[b],
                                      gsem.at[b]).wait()
                pltpu.async_copy(rows_v.at[b], acc_sp.at[cidx.at[w]],
                                 ssem.at[b], add=True)

                @pl.when(w + 1 < nw)
                def _():
                    @pl.when(w >= 1)
                    def _():
                        pltpu.make_async_copy(
                            rows_v.at[1 - b], acc_sp.at[cidx.at[w - 1]],
                            ssem.at[1 - b]).wait()

                    pltpu.async_copy(hs_hbm.at[ridx.at[w + 1]],
                                     rows_v.at[1 - b], gsem.at[1 - b])

                return carry

            lax.fori_loop(0, nw, win, 0)

            @pl.when(nw >= 2)
            def _():
                pltpu.make_async_copy(
                    rows_v.at[lax.rem(nw - 2, 2)],
                    acc_sp.at[cidx.at[nw - 2]],
                    ssem.at[lax.rem(nw - 2, 2)]).wait()

            pltpu.make_async_copy(
                rows_v.at[lax.rem(nw - 1, 2)],
                acc_sp.at[cidx.at[nw - 1]],
                ssem.at[lax.rem(nw - 1, 2)]).wait()

    plsc.subcore_barrier()
    pltpu.sync_copy(acc_sp.at[pl.ds(sid * STR1, STR1)],
                    acc_hbm.at[cid, pl.ds(sid * STR1, STR1)])


def _scb(hs, r, c, cnt, z2):
    k = functools.partial(
        pl.kernel, _scb_body,
        out_type=jax.ShapeDtypeStruct((NC, NEXT, D), F32),
        mesh=_mesh(),
        compiler_params=pltpu.CompilerParams(needs_layout_passes=False),
        scratch_types=[
            pltpu.VMEM((NWINC // 2, WIN), I32),
            pltpu.VMEM((NWINC // 2, WIN), I32),
            pltpu.VMEM((128,), F32),
            pltpu.VMEM((2, WIN, D), F32),
            pltpu.VMEM_SHARED((NEXT, D), F32),
            pltpu.SemaphoreType.DMA((2,)),
            pltpu.SemaphoreType.DMA((2,)),
        ])()
    return k(hs, r, c, cnt, z2)


def _tcm_body(xs_ref, w_ref, h_ref):
    h_ref[...] = jnp.dot(xs_ref[...], w_ref[...], preferred_element_type=F32)


_tcm = pl.pallas_call(
    _tcm_body, out_shape=jax.ShapeDtypeStruct((NEXT, D), F32))


def _tcs_body(h_ref, dega_ref, degb_ref, a_ref, hs_ref, dinv_ref):
    a = a_ref[...]
    deg = dega_ref[...] + degb_ref[...] + a
    dinv = a * lax.rsqrt(jnp.maximum(deg, 1e-12))
    hs_ref[...] = h_ref[...] * dinv
    dinv_ref[...] = dinv


_tcs = pl.pallas_call(
    _tcs_body,
    out_shape=(jax.ShapeDtypeStruct((NEXT, D), F32),
               jax.ShapeDtypeStruct((NEXT, 1), F32)))


def _tcc_body(k, acc_ref, hs_ref, dinv_ref, a_ref, p_ref, b_ref,
              xs_ref, anew_ref, feat_ref):
    dinv = dinv_ref[...]
    z = dinv * (acc_ref[0] + acc_ref[1] + hs_ref[...]) + b_ref[...]
    z = jnp.where(z >= 0, z, 0.01 * z)
    pv = p_ref[...]
    pn = pv * lax.rsqrt(jnp.sum(pv * pv))
    s = jnp.tanh(jnp.sum(z * pn, axis=1, keepdims=True))
    am = a_ref[...]
    smask = jnp.where(am > 0, s, -2.0)
    key = lax.bitcast_convert_type(smask, I32)
    key = key ^ ((key >> 31) & jnp.int32(0x7FFFFFFF))

    cand = lax.broadcasted_iota(I32, (1, 256), 1)
    bi = lax.broadcasted_iota(I32, (256, 256), 0)
    bj = lax.broadcasted_iota(I32, (256, 256), 1)
    ge_mat = jnp.where(bi >= bj, 1.0, 0.0).astype(F32)
    kf = jnp.float32(k)
    CH, CR = 8, NEXT // 8

    krem = kf
    tval = jnp.int32(0)
    cnt_eq = jnp.float32(0)
    for lvl in range(4):
        sh = 24 - 8 * lvl
        cands = cand - 128 if lvl == 0 else cand
        hist = jnp.zeros((1, 256), F32)
        for g in range(CH):
            kb = lax.slice(key, (g * CR, 0), ((g + 1) * CR, 1))
            b = kb >> 24 if lvl == 0 else (kb >> sh) & 255
            ok = b == cands
            if lvl > 0:
                ok = ok & ((kb >> (sh + 8)) == (tval >> (sh + 8)))
            hist = hist + jnp.sum(jnp.where(ok, 1.0, 0.0),
                                  axis=0, keepdims=True)
        cnt_ge = jnp.dot(hist, ge_mat, preferred_element_type=F32)
        sel = cnt_ge >= krem
        cpos = jnp.max(jnp.where(sel, cand, -1000))
        byte = cpos - 128 if lvl == 0 else cpos
        krem = krem - jnp.sum(jnp.where(cand > cpos, hist, 0.0))
        tval = tval | (byte << sh)
        if lvl == 3:
            cnt_eq = jnp.sum(jnp.where(cand == cpos, hist, 0.0))

    thr = tval
    gt = key > thr
    eq = key == thr
    rneed = krem.astype(I32)
    idx = lax.broadcasted_iota(I32, (NEXT, 1), 0)

    def tie_search(_):
        def idx_body(_, lohi):
            lo, hi = lohi
            mid = (lo + hi) // 2
            cnt = jnp.sum((eq & (idx < mid)).astype(I32))
            return (jnp.where(cnt >= rneed, lo, mid + 1),
                    jnp.where(cnt >= rneed, mid, hi))

        _, cut = lax.fori_loop(0, 15, idx_body, (jnp.int32(0), jnp.int32(16384)))
        return cut

    cut = lax.cond(rneed == cnt_eq.astype(I32),
                   lambda _: jnp.int32(NEXT), tie_search, 0)
    anew = (gt | (eq & (idx < cut))).astype(F32)
    zs = z * s
    xs_ref[...] = zs * anew
    anew_ref[...] = anew
    mx = jnp.max(jnp.where(anew > 0, zs, -1e30), axis=0, keepdims=True)
    mean = jnp.sum(zs * anew, axis=0, keepdims=True) * (1.0 / k)
    feat_ref[...] = jnp.concatenate([mx, mean], axis=0)


def _tcc(k):
    return pl.pallas_call(
        functools.partial(_tcc_body, k),
        out_shape=(jax.ShapeDtypeStruct((NEXT, D), F32),
                   jax.ShapeDtypeStruct((NEXT, 1), F32),
                   jax.ShapeDtypeStruct((2, D), F32)),
        compiler_params=pltpu.CompilerParams(
            vmem_limit_bytes=100 * 1024 * 1024))


def kernel(x, edge_index, batch, W1, b1, W2, b2, W3, b3, p1, p2, p3):
    pad = CAP - EPW
    dump = (N + (jnp.arange(NW * pad, dtype=I32) % NDUMP)).astype(I32)
    dump = dump.reshape(NW, pad)

    def shard(e):
        return jnp.concatenate([e.astype(I32).reshape(NW, EPW), dump],
                               axis=1).reshape(NW, NWINC, WIN)

    r = shard(edge_index[0])
    c = shard(edge_index[1])
    cnt = jnp.zeros((NC, 128), F32).at[:, :NS].set(float(EPW))
    a = jnp.concatenate([jnp.ones((N,), F32), jnp.zeros((NDUMP,), F32)])
    xs = jnp.zeros((NEXT, D), F32).at[:N].set(x)
    z1 = jnp.zeros((NEXT,), F32)
    z2 = jnp.zeros((NEXT, D), F32)

    feats = []
    for (W, b, p, k) in ((W1, b1, p1, 5000), (W2, b2, p2, 2500),
                         (W3, b3, p3, 1250)):
        h = _tcm(xs, W)
        r, c, cnt, d0, d1 = _scd(r, c, cnt, a, z1)
        hs, dinv = _tcs(h, d0.reshape(NEXT, 1), d1.reshape(NEXT, 1),
                        a.reshape(NEXT, 1))
        acc = _scb(hs, r, c, cnt, z2)
        xs, a_col, feat = _tcc(k)(acc, hs, dinv,
                                  a.reshape(NEXT, 1), p.reshape(1, D),
                                  b.reshape(1, D))
        a = a_col.reshape(NEXT)
        feats.append(feat.reshape(1, 2 * D))
    return jnp.concatenate(feats, axis=1)


# speedup vs baseline: 1.0514x; 1.0514x over previous
import functools

import jax
import jax.numpy as jnp
from jax import lax
from jax.experimental import pallas as pl
from jax.experimental.pallas import tpu as pltpu
from jax.experimental.pallas import tpu_sc as plsc

N = 10000
NEXT = 10240
NDUMP = NEXT - N
D = 128
E = 320000
NC, NS = 2, 16
NW = NC * NS
WIN = 128
NWINC = 96
EPW = E // NW
CAP = NWINC * WIN
STR1 = NEXT // NS
F32 = jnp.float32
I32 = jnp.int32


def _mesh():
    return plsc.VectorSubcoreMesh(
        core_axis_name="c", subcore_axis_name="s", num_cores=NC, num_subcores=NS)


def _scd_body(r_hbm, c_hbm, cnt_hbm, a_hbm, z1_hbm,
              ro_hbm, co_hbm, cnto_hbm, d0_hbm, d1_hbm,
              a_v, ridx, cidx, rout, cout, actv, cntv, cnto_v, iota_v,
              deg_sp, cnt_sp):
    cid = lax.axis_index("c")
    sid = lax.axis_index("s")
    wid = sid * NC + cid
    lane = lax.iota(I32, 16)

    @pl.when(sid == 0)
    def _():
        pltpu.sync_copy(z1_hbm.at[pl.ds(0, 128)], cnt_sp)

    pltpu.sync_copy(z1_hbm.at[pl.ds(sid * STR1, STR1)],
                    deg_sp.at[pl.ds(sid * STR1, STR1)])
    pltpu.sync_copy(a_hbm, a_v)
    pltpu.sync_copy(cnt_hbm.at[cid], cntv)
    pltpu.sync_copy(r_hbm.at[wid], ridx)
    pltpu.sync_copy(c_hbm.at[wid], cidx)
    iota_v[...] = lane
    plsc.subcore_barrier()

    mycnt = jnp.sum(jnp.where(lane == sid, cntv[pl.ds(0, 16)], 0.0)).astype(I32)
    nwin = (mycnt + (WIN - 1)) >> 7

    def win(w, off):
        for j in range(WIN // 16):
            rv = ridx[w, pl.ds(j * 16, 16)]
            cv = cidx[w, pl.ds(j * 16, 16)]
            ar = plsc.load_gather(a_v, [rv])
            ac = plsc.load_gather(a_v, [cv])
            act = (ar * ac) > 0.5
            acti = act.astype(I32)
            pos = off + plsc.cumsum(acti) - 1
            plsc.store_scatter(rout, [pos >> 7, pos & (WIN - 1)], rv, mask=act)
            plsc.store_scatter(cout, [pos >> 7, pos & (WIN - 1)], cv, mask=act)
            off = off + jnp.sum(acti)
            actv[pl.ds(j * 16, 16)] = jnp.where(act, 1.0, 0.0).astype(F32)
        pltpu.sync_copy(actv, deg_sp.at[cidx.at[w]], add=True)
        return off

    off = lax.fori_loop(0, nwin, win, jnp.int32(0))

    for j in range(WIN // 16):
        pos = off + j * 16 + lane
        dmp = N + lax.rem((wid * 61 + j) * 16 + lane, jnp.full((16,), NDUMP, I32))
        plsc.store_scatter(rout, [pos >> 7, pos & (WIN - 1)], dmp)
        plsc.store_scatter(cout, [pos >> 7, pos & (WIN - 1)], dmp)

    pltpu.sync_copy(rout, ro_hbm.at[wid])
    pltpu.sync_copy(cout, co_hbm.at[wid])
    cnto_v[...] = jnp.where(lane == sid, off.astype(F32), 0.0)
    pltpu.sync_copy(cnto_v, cnt_sp.at[iota_v], add=True)
    plsc.subcore_barrier()

    @pl.when(cid == 0)
    def _():
        pltpu.sync_copy(deg_sp.at[pl.ds(sid * STR1, STR1)],
                        d0_hbm.at[pl.ds(sid * STR1, STR1)])

    @pl.when(cid == 1)
    def _():
        pltpu.sync_copy(deg_sp.at[pl.ds(sid * STR1, STR1)],
                        d1_hbm.at[pl.ds(sid * STR1, STR1)])

    @pl.when(sid == 0)
    def _():
        pltpu.sync_copy(cnt_sp, cnto_hbm.at[cid])


def _scd(r, c, cnt, a, z1):
    k = functools.partial(
        pl.kernel, _scd_body,
        out_type=(jax.ShapeDtypeStruct((NW, NWINC, WIN), I32),
                  jax.ShapeDtypeStruct((NW, NWINC, WIN), I32),
                  jax.ShapeDtypeStruct((NC, 128), F32),
                  jax.ShapeDtypeStruct((NEXT,), F32),
                  jax.ShapeDtypeStruct((NEXT,), F32)),
        mesh=_mesh(),
        compiler_params=pltpu.CompilerParams(needs_layout_passes=False),
        scratch_types=[
            pltpu.VMEM((NEXT,), F32),
            pltpu.VMEM((NWINC, WIN), I32),
            pltpu.VMEM((NWINC, WIN), I32),
            pltpu.VMEM((NWINC, WIN), I32),
            pltpu.VMEM((NWINC, WIN), I32),
            pltpu.VMEM((WIN,), F32),
            pltpu.VMEM((128,), F32),
            pltpu.VMEM((NS,), F32),
            pltpu.VMEM((16,), I32),
            pltpu.VMEM_SHARED((NEXT,), F32),
            pltpu.VMEM_SHARED((128,), F32),
        ])()
    return k(r, c, cnt, a, z1)


def _scb_body(hs_hbm, r_hbm, c_hbm, cnt_hbm, z2_hbm, acc_hbm,
              ridx, cidx, cntv, rows_v, acc_sp):
    cid = lax.axis_index("c")
    sid = lax.axis_index("s")
    wid = sid * NC + cid
    lane = lax.iota(I32, 16)
    pltpu.sync_copy(z2_hbm.at[pl.ds(sid * STR1, STR1)],
                    acc_sp.at[pl.ds(sid * STR1, STR1)])
    pltpu.sync_copy(cnt_hbm.at[cid], cntv)
    plsc.subcore_barrier()

    mycnt = jnp.sum(jnp.where(lane == sid, cntv[pl.ds(0, 16)], 0.0)).astype(I32)
    nwin = (mycnt + (WIN - 1)) >> 7

    CW = NWINC // 2
    for ch in range(2):
        nw = jnp.clip(nwin - ch * CW, 0, CW)

        @pl.when(nw > 0)
        def _():
            pltpu.sync_copy(r_hbm.at[wid, pl.ds(ch * CW, CW)], ridx)
            pltpu.sync_copy(c_hbm.at[wid, pl.ds(ch * CW, CW)], cidx)

            def win(w, carry):
                pltpu.sync_copy(hs_hbm.at[ridx.at[w]], rows_v)
                pltpu.sync_copy(rows_v, acc_sp.at[cidx.at[w]], add=True)
                return carry

            lax.fori_loop(0, nw, win, 0)

    plsc.subcore_barrier()
    pltpu.sync_copy(acc_sp.at[pl.ds(sid * STR1, STR1)],
                    acc_hbm.at[cid, pl.ds(sid * STR1, STR1)])


def _scb(hs, r, c, cnt, z2):
    k = functools.partial(
        pl.kernel, _scb_body,
        out_type=jax.ShapeDtypeStruct((NC, NEXT, D), F32),
        mesh=_mesh(),
        compiler_params=pltpu.CompilerParams(needs_layout_passes=False),
        scratch_types=[
            pltpu.VMEM((NWINC // 2, WIN), I32),
            pltpu.VMEM((NWINC // 2, WIN), I32),
            pltpu.VMEM((128,), F32),
            pltpu.VMEM((WIN, D), F32),
            pltpu.VMEM_SHARED((NEXT, D), F32),
        ])()
    return k(hs, r, c, cnt, z2)


def _tca_body(xs_ref, w_ref, dega_ref, degb_ref, a_ref, hs_ref, dinv_ref):
    a = a_ref[...]
    deg = dega_ref[...] + degb_ref[...] + a
    dinv = a * lax.rsqrt(jnp.maximum(deg, 1e-12))
    h = jnp.dot(xs_ref[...], w_ref[...], preferred_element_type=F32)
    hs_ref[...] = h * dinv
    dinv_ref[...] = dinv


_tca = pl.pallas_call(
    _tca_body,
    out_shape=(jax.ShapeDtypeStruct((NEXT, D), F32),
               jax.ShapeDtypeStruct((NEXT, 1), F32)))


def _tcc_body(k, acc_ref, hs_ref, dinv_ref, a_ref, p_ref, b_ref,
              xs_ref, anew_ref, feat_ref):
    dinv = dinv_ref[...]
    z = dinv * (acc_ref[0] + acc_ref[1] + hs_ref[...]) + b_ref[...]
    z = jnp.where(z >= 0, z, 0.01 * z)
    pv = p_ref[...]
    pn = pv * lax.rsqrt(jnp.sum(pv * pv))
    s = jnp.tanh(jnp.sum(z * pn, axis=1, keepdims=True))
    am = a_ref[...]
    smask = jnp.where(am > 0, s, -2.0)
    key = lax.bitcast_convert_type(smask, I32)
    key = key ^ ((key >> 31) & jnp.int32(0x7FFFFFFF))

    cand = lax.broadcasted_iota(I32, (1, 256), 1)
    bi = lax.broadcasted_iota(I32, (256, 256), 0)
    bj = lax.broadcasted_iota(I32, (256, 256), 1)
    ge_mat = jnp.where(bi >= bj, 1.0, 0.0).astype(F32)
    kf = jnp.float32(k)
    CH, CR = 8, NEXT // 8

    krem = kf
    tval = jnp.int32(0)
    cnt_eq = jnp.float32(0)
    for lvl in range(4):
        sh = 24 - 8 * lvl
        cands = cand - 128 if lvl == 0 else cand
        hist = jnp.zeros((1, 256), F32)
        for g in range(CH):
            kb = lax.slice(key, (g * CR, 0), ((g + 1) * CR, 1))
            b = kb >> 24 if lvl == 0 else (kb >> sh) & 255
            ok = b == cands
            if lvl > 0:
                ok = ok & ((kb >> (sh + 8)) == (tval >> (sh + 8)))
            hist = hist + jnp.sum(jnp.where(ok, 1.0, 0.0),
                                  axis=0, keepdims=True)
        cnt_ge = jnp.dot(hist, ge_mat, preferred_element_type=F32)
        sel = cnt_ge >= krem
        cpos = jnp.max(jnp.where(sel, cand, -1000))
        byte = cpos - 128 if lvl == 0 else cpos
        krem = krem - jnp.sum(jnp.where(cand > cpos, hist, 0.0))
        tval = tval | (byte << sh)
        if lvl == 3:
            cnt_eq = jnp.sum(jnp.where(cand == cpos, hist, 0.0))

    thr = tval
    gt = key > thr
    eq = key == thr
    rneed = krem.astype(I32)
    idx = lax.broadcasted_iota(I32, (NEXT, 1), 0)

    def tie_search(_):
        def idx_body(_, lohi):
            lo, hi = lohi
            mid = (lo + hi) // 2
            cnt = jnp.sum((eq & (idx < mid)).astype(I32))
            return (jnp.where(cnt >= rneed, lo, mid + 1),
                    jnp.where(cnt >= rneed, mid, hi))

        _, cut = lax.fori_loop(0, 15, idx_body, (jnp.int32(0), jnp.int32(16384)))
        return cut

    cut = lax.cond(rneed == cnt_eq.astype(I32),
                   lambda _: jnp.int32(NEXT), tie_search, 0)
    anew = (gt | (eq & (idx < cut))).astype(F32)
    zs = z * s
    xs_ref[...] = zs * anew
    anew_ref[...] = anew
    mx = jnp.max(jnp.where(anew > 0, zs, -1e30), axis=0, keepdims=True)
    mean = jnp.sum(zs * anew, axis=0, keepdims=True) * (1.0 / k)
    feat_ref[...] = jnp.concatenate([mx, mean], axis=0)


def _tcc(k):
    return pl.pallas_call(
        functools.partial(_tcc_body, k),
        out_shape=(jax.ShapeDtypeStruct((NEXT, D), F32),
                   jax.ShapeDtypeStruct((NEXT, 1), F32),
                   jax.ShapeDtypeStruct((2, D), F32)),
        compiler_params=pltpu.CompilerParams(
            vmem_limit_bytes=100 * 1024 * 1024))


def kernel(x, edge_index, batch, W1, b1, W2, b2, W3, b3, p1, p2, p3):
    pad = CAP - EPW
    dump = (N + (jnp.arange(NW * pad, dtype=I32) % NDUMP)).astype(I32)
    dump = dump.reshape(NW, pad)

    def shard(e):
        return jnp.concatenate([e.astype(I32).reshape(NW, EPW), dump],
                               axis=1).reshape(NW, NWINC, WIN)

    r = shard(edge_index[0])
    c = shard(edge_index[1])
    cnt = jnp.zeros((NC, 128), F32).at[:, :NS].set(float(EPW))
    a = jnp.concatenate([jnp.ones((N,), F32), jnp.zeros((NDUMP,), F32)])
    xs = jnp.zeros((NEXT, D), F32).at[:N].set(x)
    z1 = jnp.zeros((NEXT,), F32)
    z2 = jnp.zeros((NEXT, D), F32)

    feats = []
    for (W, b, p, k) in ((W1, b1, p1, 5000), (W2, b2, p2, 2500),
                         (W3, b3, p3, 1250)):
        r, c, cnt, d0, d1 = _scd(r, c, cnt, a, z1)
        hs, dinv = _tca(xs, W, d0.reshape(NEXT, 1), d1.reshape(NEXT, 1),
                        a.reshape(NEXT, 1))
        acc = _scb(hs, r, c, cnt, z2)
        xs, a_col, feat = _tcc(k)(acc, hs, dinv,
                                  a.reshape(NEXT, 1), p.reshape(1, D),
                                  b.reshape(1, D))
        a = a_col.reshape(NEXT)
        feats.append(feat.reshape(1, 2 * D))
    return jnp.concatenate(feats, axis=1)


# speedup vs baseline: 1.0585x; 1.0068x over previous
import functools

import jax
import jax.numpy as jnp
from jax import lax
from jax.experimental import pallas as pl
from jax.experimental.pallas import tpu as pltpu
from jax.experimental.pallas import tpu_sc as plsc

N = 10000
NEXT = 10240
NDUMP = NEXT - N
D = 128
E = 320000
NC, NS = 2, 16
NW = NC * NS
WIN = 128
NWINC = 96
EPW = E // NW
CAP = NWINC * WIN
STR1 = NEXT // NS
F32 = jnp.float32
I32 = jnp.int32


def _mesh():
    return plsc.VectorSubcoreMesh(
        core_axis_name="c", subcore_axis_name="s", num_cores=NC, num_subcores=NS)


def _scd_body(r_hbm, c_hbm, cnt_hbm, a_hbm, z1_hbm,
              ro_hbm, co_hbm, cnto_hbm, d0_hbm, d1_hbm,
              a_v, ridx, cidx, rout, cout, actv, cntv, cnto_v, iota_v,
              deg_sp, cnt_sp):
    cid = lax.axis_index("c")
    sid = lax.axis_index("s")
    wid = sid * NC + cid
    lane = lax.iota(I32, 16)

    @pl.when(sid == 0)
    def _():
        pltpu.sync_copy(z1_hbm.at[pl.ds(0, 128)], cnt_sp)

    pltpu.sync_copy(z1_hbm.at[pl.ds(sid * STR1, STR1)],
                    deg_sp.at[pl.ds(sid * STR1, STR1)])
    pltpu.sync_copy(a_hbm, a_v)
    pltpu.sync_copy(cnt_hbm.at[cid], cntv)
    pltpu.sync_copy(r_hbm.at[wid], ridx)
    pltpu.sync_copy(c_hbm.at[wid], cidx)
    iota_v[...] = lane
    plsc.subcore_barrier()

    mycnt = jnp.sum(jnp.where(lane == sid, cntv[pl.ds(0, 16)], 0.0)).astype(I32)
    nwin = (mycnt + (WIN - 1)) >> 7

    def win(w, off):
        for j in range(WIN // 16):
            rv = ridx[w, pl.ds(j * 16, 16)]
            cv = cidx[w, pl.ds(j * 16, 16)]
            ar = plsc.load_gather(a_v, [rv])
            ac = plsc.load_gather(a_v, [cv])
            act = (ar * ac) > 0.5
            acti = act.astype(I32)
            pos = off + plsc.cumsum(acti) - 1
            plsc.store_scatter(rout, [pos >> 7, pos & (WIN - 1)], rv, mask=act)
            plsc.store_scatter(cout, [pos >> 7, pos & (WIN - 1)], cv, mask=act)
            off = off + jnp.sum(acti)
            actv[pl.ds(j * 16, 16)] = jnp.where(act, 1.0, 0.0).astype(F32)
        pltpu.sync_copy(actv, deg_sp.at[cidx.at[w]], add=True)
        return off

    off = lax.fori_loop(0, nwin, win, jnp.int32(0))

    for j in range(WIN // 16):
        pos = off + j * 16 + lane
        dmp = N + lax.rem((wid * 61 + j) * 16 + lane, jnp.full((16,), NDUMP, I32))
        plsc.store_scatter(rout, [pos >> 7, pos & (WIN - 1)], dmp)
        plsc.store_scatter(cout, [pos >> 7, pos & (WIN - 1)], dmp)

    pltpu.sync_copy(rout, ro_hbm.at[wid])
    pltpu.sync_copy(cout, co_hbm.at[wid])
    cnto_v[...] = jnp.where(lane == sid, off.astype(F32), 0.0)
    pltpu.sync_copy(cnto_v, cnt_sp.at[iota_v], add=True)
    plsc.subcore_barrier()

    @pl.when(cid == 0)
    def _():
        pltpu.sync_copy(deg_sp.at[pl.ds(sid * STR1, STR1)],
                        d0_hbm.at[pl.ds(sid * STR1, STR1)])

    @pl.when(cid == 1)
    def _():
        pltpu.sync_copy(deg_sp.at[pl.ds(sid * STR1, STR1)],
                        d1_hbm.at[pl.ds(sid * STR1, STR1)])

    @pl.when(sid == 0)
    def _():
        pltpu.sync_copy(cnt_sp, cnto_hbm.at[cid])


def _scd(r, c, cnt, a, z1):
    k = functools.partial(
        pl.kernel, _scd_body,
        out_type=(jax.ShapeDtypeStruct((NW, NWINC, WIN), I32),
                  jax.ShapeDtypeStruct((NW, NWINC, WIN), I32),
                  jax.ShapeDtypeStruct((NC, 128), F32),
                  jax.ShapeDtypeStruct((NEXT,), F32),
                  jax.ShapeDtypeStruct((NEXT,), F32)),
        mesh=_mesh(),
        compiler_params=pltpu.CompilerParams(needs_layout_passes=False),
        scratch_types=[
            pltpu.VMEM((NEXT,), F32),
            pltpu.VMEM((NWINC, WIN), I32),
            pltpu.VMEM((NWINC, WIN), I32),
            pltpu.VMEM((NWINC, WIN), I32),
            pltpu.VMEM((NWINC, WIN), I32),
            pltpu.VMEM((WIN,), F32),
            pltpu.VMEM((128,), F32),
            pltpu.VMEM((NS,), F32),
            pltpu.VMEM((16,), I32),
            pltpu.VMEM_SHARED((NEXT,), F32),
            pltpu.VMEM_SHARED((128,), F32),
        ])()
    return k(r, c, cnt, a, z1)


def _scb_body(hs_hbm, r_hbm, c_hbm, cnt_hbm, z2_hbm, acc_hbm,
              ridx, cidx, cntv, rows_v, acc_sp):
    cid = lax.axis_index("c")
    sid = lax.axis_index("s")
    wid = sid * NC + cid
    lane = lax.iota(I32, 16)

    @pl.when(cid == 0)
    def _():
        pltpu.sync_copy(hs_hbm.at[pl.ds(sid * STR1, STR1)],
                        acc_sp.at[pl.ds(sid * STR1, STR1)])

    @pl.when(cid == 1)
    def _():
        pltpu.sync_copy(z2_hbm.at[pl.ds(sid * STR1, STR1)],
                        acc_sp.at[pl.ds(sid * STR1, STR1)])

    pltpu.sync_copy(cnt_hbm.at[cid], cntv)
    plsc.subcore_barrier()

    mycnt = jnp.sum(jnp.where(lane == sid, cntv[pl.ds(0, 16)], 0.0)).astype(I32)
    nwin = (mycnt + (WIN - 1)) >> 7

    CW = NWINC // 2
    for ch in range(2):
        nw = jnp.clip(nwin - ch * CW, 0, CW)

        @pl.when(nw > 0)
        def _():
            pltpu.sync_copy(r_hbm.at[wid, pl.ds(ch * CW, CW)], ridx)
            pltpu.sync_copy(c_hbm.at[wid, pl.ds(ch * CW, CW)], cidx)

            def win(w, carry):
                pltpu.sync_copy(hs_hbm.at[ridx.at[w]], rows_v)
                pltpu.sync_copy(rows_v, acc_sp.at[cidx.at[w]], add=True)
                return carry

            lax.fori_loop(0, nw, win, 0)

    plsc.subcore_barrier()
    pltpu.sync_copy(acc_sp.at[pl.ds(sid * STR1, STR1)],
                    acc_hbm.at[cid, pl.ds(sid * STR1, STR1)])


def _scb(hs, r, c, cnt, z2):
    k = functools.partial(
        pl.kernel, _scb_body,
        out_type=jax.ShapeDtypeStruct((NC, NEXT, D), F32),
        mesh=_mesh(),
        compiler_params=pltpu.CompilerParams(needs_layout_passes=False),
        scratch_types=[
            pltpu.VMEM((NWINC // 2, WIN), I32),
            pltpu.VMEM((NWINC // 2, WIN), I32),
            pltpu.VMEM((128,), F32),
            pltpu.VMEM((WIN, D), F32),
            pltpu.VMEM_SHARED((NEXT, D), F32),
        ])()
    return k(hs, r, c, cnt, z2)


def _tca_body(xs_ref, w_ref, dega_ref, degb_ref, a_ref, hs_ref, dinv_ref):
    a = a_ref[...]
    deg = dega_ref[...] + degb_ref[...] + a
    dinv = a * lax.rsqrt(jnp.maximum(deg, 1e-12))
    h = jnp.dot(xs_ref[...], w_ref[...], preferred_element_type=F32)
    hs_ref[...] = h * dinv
    dinv_ref[...] = dinv


_tca = pl.pallas_call(
    _tca_body,
    out_shape=(jax.ShapeDtypeStruct((NEXT, D), F32),
               jax.ShapeDtypeStruct((NEXT, 1), F32)))


def _tcc_body(k, acc_ref, dinv_ref, a_ref, p_ref, b_ref,
              xs_ref, anew_ref, feat_ref):
    dinv = dinv_ref[...]
    z = dinv * (acc_ref[0] + acc_ref[1]) + b_ref[...]
    z = jnp.where(z >= 0, z, 0.01 * z)
    pv = p_ref[...]
    pn = pv * lax.rsqrt(jnp.sum(pv * pv))
    s = jnp.tanh(jnp.sum(z * pn, axis=1, keepdims=True))
    am = a_ref[...]
    smask = jnp.where(am > 0, s, -2.0)
    key = lax.bitcast_convert_type(smask, I32)
    key = key ^ ((key >> 31) & jnp.int32(0x7FFFFFFF))

    cand = lax.broadcasted_iota(I32, (1, 256), 1)
    bi = lax.broadcasted_iota(I32, (256, 256), 0)
    bj = lax.broadcasted_iota(I32, (256, 256), 1)
    ge_mat = jnp.where(bi >= bj, 1.0, 0.0).astype(F32)
    kf = jnp.float32(k)
    CH, CR = 8, NEXT // 8

    krem = kf
    tval = jnp.int32(0)
    cnt_eq = jnp.float32(0)
    for lvl in range(4):
        sh = 24 - 8 * lvl
        cands = cand - 128 if lvl == 0 else cand
        hist = jnp.zeros((1, 256), F32)
        for g in range(CH):
            kb = lax.slice(key, (g * CR, 0), ((g + 1) * CR, 1))
            b = kb >> 24 if lvl == 0 else (kb >> sh) & 255
            ok = b == cands
            if lvl > 0:
                ok = ok & ((kb >> (sh + 8)) == (tval >> (sh + 8)))
            hist = hist + jnp.sum(jnp.where(ok, 1.0, 0.0),
                                  axis=0, keepdims=True)
        cnt_ge = jnp.dot(hist, ge_mat, preferred_element_type=F32)
        sel = cnt_ge >= krem
        cpos = jnp.max(jnp.where(sel, cand, -1000))
        byte = cpos - 128 if lvl == 0 else cpos
        krem = krem - jnp.sum(jnp.where(cand > cpos, hist, 0.0))
        tval = tval | (byte << sh)
        if lvl == 3:
            cnt_eq = jnp.sum(jnp.where(cand == cpos, hist, 0.0))

    thr = tval
    gt = key > thr
    eq = key == thr
    rneed = krem.astype(I32)
    idx = lax.broadcasted_iota(I32, (NEXT, 1), 0)

    def tie_search(_):
        def idx_body(_, lohi):
            lo, hi = lohi
            mid = (lo + hi) // 2
            cnt = jnp.sum((eq & (idx < mid)).astype(I32))
            return (jnp.where(cnt >= rneed, lo, mid + 1),
                    jnp.where(cnt >= rneed, mid, hi))

        _, cut = lax.fori_loop(0, 15, idx_body, (jnp.int32(0), jnp.int32(16384)))
        return cut

    cut = lax.cond(rneed == cnt_eq.astype(I32),
                   lambda _: jnp.int32(NEXT), tie_search, 0)
    anew = (gt | (eq & (idx < cut))).astype(F32)
    zs = z * s
    xs_ref[...] = zs * anew
    anew_ref[...] = anew
    mx = jnp.max(jnp.where(anew > 0, zs, -1e30), axis=0, keepdims=True)
    mean = jnp.sum(zs * anew, axis=0, keepdims=True) * (1.0 / k)
    feat_ref[...] = jnp.concatenate([mx, mean], axis=0)


def _tcc(k):
    return pl.pallas_call(
        functools.partial(_tcc_body, k),
        out_shape=(jax.ShapeDtypeStruct((NEXT, D), F32),
                   jax.ShapeDtypeStruct((NEXT, 1), F32),
                   jax.ShapeDtypeStruct((2, D), F32)),
        compiler_params=pltpu.CompilerParams(
            vmem_limit_bytes=100 * 1024 * 1024))


def kernel(x, edge_index, batch, W1, b1, W2, b2, W3, b3, p1, p2, p3):
    pad = CAP - EPW
    dump = (N + (jnp.arange(NW * pad, dtype=I32) % NDUMP)).astype(I32)
    dump = dump.reshape(NW, pad)

    def shard(e):
        return jnp.concatenate([e.astype(I32).reshape(NW, EPW), dump],
                               axis=1).reshape(NW, NWINC, WIN)

    r = shard(edge_index[0])
    c = shard(edge_index[1])
    cnt = jnp.zeros((NC, 128), F32).at[:, :NS].set(float(EPW))
    a = jnp.concatenate([jnp.ones((N,), F32), jnp.zeros((NDUMP,), F32)])
    xs = jnp.zeros((NEXT, D), F32).at[:N].set(x)
    z1 = jnp.zeros((NEXT,), F32)
    z2 = jnp.zeros((NEXT, D), F32)

    feats = []
    for (W, b, p, k) in ((W1, b1, p1, 5000), (W2, b2, p2, 2500),
                         (W3, b3, p3, 1250)):
        r, c, cnt, d0, d1 = _scd(r, c, cnt, a, z1)
        hs, dinv = _tca(xs, W, d0.reshape(NEXT, 1), d1.reshape(NEXT, 1),
                        a.reshape(NEXT, 1))
        acc = _scb(hs, r, c, cnt, z2)
        xs, a_col, feat = _tcc(k)(acc, dinv,
                                  a.reshape(NEXT, 1), p.reshape(1, D),
                                  b.reshape(1, D))
        a = a_col.reshape(NEXT)
        feats.append(feat.reshape(1, 2 * D))
    return jnp.concatenate(feats, axis=1)
